# baseline (speedup 1.0000x reference)
import functools

import jax
import jax.numpy as jnp
from jax.experimental import pallas as pl
from jax.experimental.pallas import tpu as pltpu
from jax.experimental.pallas import tpu_sc as plsc

MAX_IDX = 400000
SPLIT = 204800
DIM = 64
DM = 1024
WINDOW = 128
REG = 16

T_COLS = 20480
T_STEPS = SPLIT // T_COLS


def _tc_transpose(tabT):
    def body(a_ref, b_ref, o_ref):
        o_ref[:, :DIM] = a_ref[...].T
        o_ref[:, DIM:] = b_ref[...].T

    return pl.pallas_call(
        body,
        grid=(T_STEPS,),
        in_specs=[
            pl.BlockSpec((DIM, T_COLS), lambda i: (0, i)),
            pl.BlockSpec((DIM, T_COLS), lambda i: (0, i + T_STEPS)),
        ],
        out_specs=pl.BlockSpec((T_COLS, 2 * DIM), lambda i: (i, 0)),
        out_shape=jax.ShapeDtypeStruct((SPLIT, 2 * DIM), jnp.float32),
        compiler_params=pltpu.CompilerParams(
            dimension_semantics=("parallel",),
        ),
    )(tabT, tabT)


def _sc_hash_gather(x2, p2, t2):
    n = x2.shape[1]
    mesh = plsc.VectorSubcoreMesh(core_axis_name="c", subcore_axis_name="s")

    @functools.partial(
        pl.kernel,
        out_type=jax.ShapeDtypeStruct((n, 2 * DIM), jnp.float32),
        mesh=mesh,
        scratch_types=[pltpu.VMEM((1, WINDOW), jnp.int32)],
    )
    def k(x_hbm, p_hbm, t_hbm, o_hbm, idx_s):
        def body(x_v, p_v, o_v):
            @pl.loop(0, WINDOW, step=REG)
            def _(c):
                slc = (pl.ds(0, 1), pl.ds(c, REG))
                h = x_v.at[*slc][...] * 3 + p_v.at[*slc][...]
                idx_s.at[*slc][...] = jnp.where(h >= SPLIT, h - SPLIT, h)

            pltpu.sync_copy(t_hbm.at[idx_s.at[0]], o_v)

        pltpu.emit_pipeline(
            body,
            grid=(n // WINDOW,),
            in_specs=[
                pl.BlockSpec((1, WINDOW), lambda i: (0, i)),
                pl.BlockSpec((1, WINDOW), lambda i: (0, i)),
            ],
            out_specs=[pl.BlockSpec((WINDOW, 2 * DIM), lambda i: (i, 0))],
            core_axis_name=("c", "s"),
            dimension_semantics=(pltpu.PARALLEL,),
        )(x_hbm, p_hbm, o_hbm)

    return k(x2, p2, t2)


def _tc_project(emb, xc, pc, w):
    n = emb.shape[0]
    rows = 2048

    def body(e_ref, x_ref, p_ref, w_ref, o_ref):
        h = x_ref[...] * 3 + p_ref[...]
        e = jnp.where(h >= SPLIT, e_ref[:, DIM:], e_ref[:, :DIM])
        o_ref[...] = jax.lax.dot_general(
            e, w_ref[...],
            (((1,), (1,)), ((), ())),
            preferred_element_type=jnp.float32,
        )

    return pl.pallas_call(
        body,
        grid=(n // rows,),
        in_specs=[
            pl.BlockSpec((rows, 2 * DIM), lambda i: (i, 0)),
            pl.BlockSpec((rows, 1), lambda i: (i, 0)),
            pl.BlockSpec((rows, 1), lambda i: (i, 0)),
            pl.BlockSpec((DM, DIM), lambda i: (0, 0)),
        ],
        out_specs=pl.BlockSpec((rows, DM), lambda i: (i, 0)),
        out_shape=jax.ShapeDtypeStruct((n, DM), jnp.float32),
        compiler_params=pltpu.CompilerParams(
            dimension_semantics=("parallel",),
        ),
    )(emb, xc, pc, w)


def kernel(x, table, W):
    b, s = x.shape
    x32 = x.astype(jnp.int32)
    prev = jnp.roll(x32, 1, axis=1).at[:, 0].set(0)
    n = b * s
    with jax.enable_x64(False):
        t2 = _tc_transpose(table.T)
        emb = _sc_hash_gather(x32.reshape(1, n), prev.reshape(1, n), t2)
        out = _tc_project(emb, x32.reshape(n, 1), prev.reshape(n, 1), W)
    return out.reshape(b, s, DM)


# speedup vs baseline: 1.0119x; 1.0119x over previous
import functools

import jax
import jax.numpy as jnp
from jax.experimental import pallas as pl
from jax.experimental.pallas import tpu as pltpu
from jax.experimental.pallas import tpu_sc as plsc

MAX_IDX = 400000
SPLIT = 204800
DIM = 64
DM = 1024
WINDOW = 128
REG = 16

T_COLS = 12800
T_STEPS = SPLIT // T_COLS


def _tc_transpose(tabT):
    def body(a_ref, b_ref, o_ref):
        o_ref[:, :DIM] = a_ref[...].T
        o_ref[:, DIM:] = b_ref[...].T

    return pl.pallas_call(
        body,
        grid=(T_STEPS,),
        in_specs=[
            pl.BlockSpec((DIM, T_COLS), lambda i: (0, i)),
            pl.BlockSpec((DIM, T_COLS), lambda i: (0, i + T_STEPS)),
        ],
        out_specs=pl.BlockSpec((T_COLS, 2 * DIM), lambda i: (i, 0)),
        out_shape=jax.ShapeDtypeStruct((SPLIT, 2 * DIM), jnp.float32),
        compiler_params=pltpu.CompilerParams(
            dimension_semantics=("parallel",),
        ),
    )(tabT, tabT)


def _sc_hash_gather(x2, p2, t2):
    n = x2.shape[1]
    mesh = plsc.VectorSubcoreMesh(core_axis_name="c", subcore_axis_name="s")

    @functools.partial(
        pl.kernel,
        out_type=jax.ShapeDtypeStruct((n, 2 * DIM), jnp.float32),
        mesh=mesh,
        scratch_types=[pltpu.VMEM((1, WINDOW), jnp.int32)],
    )
    def k(x_hbm, p_hbm, t_hbm, o_hbm, idx_s):
        def body(x_v, p_v, o_v):
            @pl.loop(0, WINDOW, step=REG)
            def _(c):
                slc = (pl.ds(0, 1), pl.ds(c, REG))
                h = x_v.at[*slc][...] * 3 + p_v.at[*slc][...]
                idx_s.at[*slc][...] = jnp.where(h >= SPLIT, h - SPLIT, h)

            pltpu.sync_copy(t_hbm.at[idx_s.at[0]], o_v)

        pltpu.emit_pipeline(
            body,
            grid=(n // WINDOW,),
            in_specs=[
                pl.BlockSpec((1, WINDOW), lambda i: (0, i)),
                pl.BlockSpec((1, WINDOW), lambda i: (0, i)),
            ],
            out_specs=[pl.BlockSpec((WINDOW, 2 * DIM), lambda i: (i, 0))],
            core_axis_name=("c", "s"),
            dimension_semantics=(pltpu.PARALLEL,),
        )(x_hbm, p_hbm, o_hbm)

    return k(x2, p2, t2)


def _tc_project(emb, xc, pc, w):
    n = emb.shape[0]
    rows = 2048

    def body(e_ref, x_ref, p_ref, w_ref, o_ref):
        h = x_ref[...] * 3 + p_ref[...]
        e = jnp.where(h >= SPLIT, e_ref[:, DIM:], e_ref[:, :DIM])
        o_ref[...] = jax.lax.dot_general(
            e, w_ref[...],
            (((1,), (1,)), ((), ())),
            preferred_element_type=jnp.float32,
        )

    return pl.pallas_call(
        body,
        grid=(n // rows,),
        in_specs=[
            pl.BlockSpec((rows, 2 * DIM), lambda i: (i, 0)),
            pl.BlockSpec((rows, 1), lambda i: (i, 0)),
            pl.BlockSpec((rows, 1), lambda i: (i, 0)),
            pl.BlockSpec((DM, DIM), lambda i: (0, 0)),
        ],
        out_specs=pl.BlockSpec((rows, DM), lambda i: (i, 0)),
        out_shape=jax.ShapeDtypeStruct((n, DM), jnp.float32),
        compiler_params=pltpu.CompilerParams(
            dimension_semantics=("parallel",),
        ),
    )(emb, xc, pc, w)


def kernel(x, table, W):
    b, s = x.shape
    x32 = x.astype(jnp.int32)
    prev = jnp.roll(x32, 1, axis=1).at[:, 0].set(0)
    n = b * s
    with jax.enable_x64(False):
        t2 = _tc_transpose(table.T)
        emb = _sc_hash_gather(x32.reshape(1, n), prev.reshape(1, n), t2)
        out = _tc_project(emb, x32.reshape(n, 1), prev.reshape(n, 1), W)
    return out.reshape(b, s, DM)
